# Initial kernel scaffold; baseline (speedup 1.0000x reference)
import jax
import jax.numpy as jnp
from jax.experimental import pallas as pl


def kernel(x, edge_index, W1_l, b1, W1_r, W2_l, b2, W2_r):
    raise NotImplementedError("write your pallas kernel here")


# speedup vs baseline: 6.7265x; 6.7265x over previous
import functools

import jax
import jax.numpy as jnp
from jax import lax
from jax.experimental import pallas as pl
from jax.experimental.pallas import tpu as pltpu
from jax.experimental.pallas import tpu_sc as plsc

N_NODES = 10000
N_EDGES = 320000
D_IN = 128
D_HID = 32

NC = 2
NS = 16
NW = NC * NS
EPW = N_EDGES // NW
CHUNK = 80
NCHUNK = EPW // CHUNK
N_PAD = 10240
RPS = N_PAD // NS


def _sc_segment_sum(table, src, dst, zeros, d):
    mesh = plsc.VectorSubcoreMesh(core_axis_name="c", subcore_axis_name="s")

    @functools.partial(
        pl.kernel,
        mesh=mesh,
        compiler_params=pltpu.CompilerParams(use_tc_tiling_on_sc=False),
        out_type=jax.ShapeDtypeStruct((NC, N_PAD, d), jnp.float32),
        scratch_types=[
            pltpu.VMEM((1, CHUNK), jnp.int32),
            pltpu.VMEM((1, CHUNK), jnp.int32),
            pltpu.VMEM((CHUNK, d), jnp.float32),
            pltpu.VMEM_SHARED((N_PAD, d), jnp.float32),
        ],
    )
    def k(tab_hbm, src_hbm, dst_hbm, z_hbm, out_hbm, srcb, dstb, rows, acc):
        cid = lax.axis_index("c")
        sid = lax.axis_index("s")
        wid = sid * NC + cid

        pltpu.sync_copy(z_hbm.at[pl.ds(sid * RPS, RPS)],
                        acc.at[pl.ds(sid * RPS, RPS)])
        plsc.subcore_barrier()

        base = wid * EPW

        @pl.loop(0, NCHUNK)
        def _(j):
            off = base + j * CHUNK
            pltpu.sync_copy(src_hbm.at[pl.ds(off, CHUNK)], srcb.at[0])
            pltpu.sync_copy(dst_hbm.at[pl.ds(off, CHUNK)], dstb.at[0])
            pltpu.sync_copy(tab_hbm.at[srcb.at[0]], rows)
            pltpu.sync_copy(rows, acc.at[dstb.at[0]], add=True)

        plsc.subcore_barrier()
        pltpu.sync_copy(acc.at[pl.ds(sid * RPS, RPS)],
                        out_hbm.at[cid].at[pl.ds(sid * RPS, RPS)])

    return k(table, src, dst, zeros)


_ROWS = 1000


def _k1_body(x_ref, wl_ref, ones_ref, wr_ref, xp_ref, xr_ref):
    xb = x_ref[...]
    xp_ref[...] = jnp.dot(xb, wl_ref[...],
                          preferred_element_type=jnp.float32) + ones_ref[...]
    xr_ref[...] = jnp.dot(xb, wr_ref[...], preferred_element_type=jnp.float32)


def _tc_project(x, wl_pad, ones_row, w1r):
    grid = N_NODES // _ROWS
    return pl.pallas_call(
        _k1_body,
        grid=(grid,),
        in_specs=[
            pl.BlockSpec((_ROWS, D_IN), lambda i: (i, 0)),
            pl.BlockSpec((D_IN, 48), lambda i: (0, 0)),
            pl.BlockSpec((1, 48), lambda i: (0, 0)),
            pl.BlockSpec((D_IN, D_HID), lambda i: (0, 0)),
        ],
        out_specs=[
            pl.BlockSpec((_ROWS, 48), lambda i: (i, 0)),
            pl.BlockSpec((_ROWS, D_HID), lambda i: (i, 0)),
        ],
        out_shape=[
            jax.ShapeDtypeStruct((N_NODES, 48), jnp.float32),
            jax.ShapeDtypeStruct((N_NODES, D_HID), jnp.float32),
        ],
    )(x, wl_pad, ones_row, w1r)


def _k2_body(agg_ref, xr_ref, b1_ref, h_ref):
    a = agg_ref[0] + agg_ref[1]
    dinv = 1.0 / jnp.clip(a[:, 32:33], 1.0, None)
    h_ref[...] = jax.nn.relu(a[:, :D_HID] * dinv + b1_ref[...] + xr_ref[...])


def _tc_hidden(agg1, xr, b1_row):
    grid = N_NODES // _ROWS
    return pl.pallas_call(
        _k2_body,
        grid=(grid,),
        in_specs=[
            pl.BlockSpec((NC, _ROWS, 48), lambda i: (0, i, 0)),
            pl.BlockSpec((_ROWS, D_HID), lambda i: (i, 0)),
            pl.BlockSpec((1, D_HID), lambda i: (0, 0)),
        ],
        out_specs=pl.BlockSpec((_ROWS, D_HID), lambda i: (i, 0)),
        out_shape=jax.ShapeDtypeStruct((N_NODES, D_HID), jnp.float32),
    )(agg1, xr, b1_row)


def _k3_body(aggh_ref, agg1_ref, h_ref, w2l_ref, b2_ref, w2r_ref, out_ref):
    deg = jnp.clip(agg1_ref[0][:, 32:33] + agg1_ref[1][:, 32:33], 1.0, None)
    m = (aggh_ref[0] + aggh_ref[1]) / deg
    out_ref[...] = (jnp.dot(m, w2l_ref[...], preferred_element_type=jnp.float32)
                    + b2_ref[...]
                    + jnp.dot(h_ref[...], w2r_ref[...],
                              preferred_element_type=jnp.float32))


def _tc_out(aggh, agg1, h, w2l, b2_row, w2r):
    grid = N_NODES // _ROWS
    return pl.pallas_call(
        _k3_body,
        grid=(grid,),
        in_specs=[
            pl.BlockSpec((NC, _ROWS, D_HID), lambda i: (0, i, 0)),
            pl.BlockSpec((NC, _ROWS, 48), lambda i: (0, i, 0)),
            pl.BlockSpec((_ROWS, D_HID), lambda i: (i, 0)),
            pl.BlockSpec((D_HID, 128), lambda i: (0, 0)),
            pl.BlockSpec((1, 128), lambda i: (0, 0)),
            pl.BlockSpec((D_HID, 128), lambda i: (0, 0)),
        ],
        out_specs=pl.BlockSpec((_ROWS, 128), lambda i: (i, 0)),
        out_shape=jax.ShapeDtypeStruct((N_NODES, 128), jnp.float32),
    )(aggh, agg1, h, w2l, b2_row, w2r)


def kernel(x, edge_index, W1_l, b1, W1_r, W2_l, b2, W2_r):
    src = edge_index[0].astype(jnp.int32)
    dst = edge_index[1].astype(jnp.int32)

    wl_pad = jnp.pad(W1_l, ((0, 0), (0, 16)))
    ones_row = jnp.zeros((1, 48), jnp.float32).at[0, 32].set(1.0)
    zeros48 = jnp.zeros((N_PAD, 48), jnp.float32)
    zeros32 = jnp.zeros((N_PAD, D_HID), jnp.float32)

    xp_aug, xr = _tc_project(x, wl_pad, ones_row, W1_r)
    agg1 = _sc_segment_sum(xp_aug, src, dst, zeros48, 48)
    h = _tc_hidden(agg1, xr, b1.reshape(1, D_HID))
    aggh = _sc_segment_sum(h, src, dst, zeros32, D_HID)
    out = _tc_out(aggh, agg1, h, W2_l, b2.reshape(1, 128), W2_r)
    return out


# speedup vs baseline: 8.8890x; 1.3215x over previous
import functools

import jax
import jax.numpy as jnp
from jax import lax
from jax.experimental import pallas as pl
from jax.experimental.pallas import tpu as pltpu
from jax.experimental.pallas import tpu_sc as plsc

N_NODES = 10000
N_EDGES = 320000
D_IN = 128
D_HID = 32

NC = 2
NS = 16
NW = NC * NS
CHUNK = 128
NCHUNK = 80
E_PAD = NW * NCHUNK * CHUNK
N_PAD = 10240
RPS = N_PAD // NS
NBUF = 4


def _sc_segment_sum(table, src2d, dst2d, zeros, d):
    mesh = plsc.VectorSubcoreMesh(core_axis_name="c", subcore_axis_name="s")

    @functools.partial(
        pl.kernel,
        mesh=mesh,
        compiler_params=pltpu.CompilerParams(use_tc_tiling_on_sc=False),
        out_type=jax.ShapeDtypeStruct((NC, N_PAD, d), jnp.float32),
        scratch_types=[
            pltpu.VMEM((NCHUNK, CHUNK), jnp.int32),
            pltpu.VMEM((NCHUNK, CHUNK), jnp.int32),
            [pltpu.VMEM((CHUNK, d), jnp.float32) for _ in range(NBUF)],
            pltpu.VMEM_SHARED((N_PAD, d), jnp.float32),
            pltpu.SemaphoreType.DMA,
            [pltpu.SemaphoreType.DMA for _ in range(NBUF)],
            [pltpu.SemaphoreType.DMA for _ in range(NBUF)],
        ],
    )
    def k(tab_hbm, src_hbm, dst_hbm, z_hbm, out_hbm,
          srcb, dstb, rows, acc, s_misc, sg, ss):
        cid = lax.axis_index("c")
        sid = lax.axis_index("s")
        wid = sid * NC + cid
        base = wid * NCHUNK

        pltpu.async_copy(src_hbm.at[pl.ds(base, NCHUNK)], srcb, s_misc)
        pltpu.async_copy(dst_hbm.at[pl.ds(base, NCHUNK)], dstb, s_misc)
        pltpu.async_copy(z_hbm.at[pl.ds(sid * RPS, RPS)],
                         acc.at[pl.ds(sid * RPS, RPS)], s_misc)
        pltpu.make_async_copy(src_hbm.at[pl.ds(base, NCHUNK)], srcb,
                              s_misc).wait()
        pltpu.make_async_copy(dst_hbm.at[pl.ds(base, NCHUNK)], dstb,
                              s_misc).wait()
        pltpu.make_async_copy(z_hbm.at[pl.ds(sid * RPS, RPS)],
                              acc.at[pl.ds(sid * RPS, RPS)], s_misc).wait()
        plsc.subcore_barrier()

        for kk in range(NBUF - 1):
            pltpu.async_copy(tab_hbm.at[srcb.at[kk]], rows[kk], sg[kk])

        def step(s, kk):
            pltpu.make_async_copy(tab_hbm.at[srcb.at[s]], rows[kk],
                                  sg[kk]).wait()
            pltpu.async_copy(rows[kk], acc.at[dstb.at[s]], ss[kk], add=True)
            kn = (kk + NBUF - 1) % NBUF
            sn = s + NBUF - 1

            @pl.when(sn < NCHUNK)
            def _():
                @pl.when(s >= 1)
                def _():
                    pltpu.make_async_copy(rows[kn], acc.at[dstb.at[s - 1]],
                                          ss[kn]).wait()
                pltpu.async_copy(tab_hbm.at[srcb.at[sn]], rows[kn], sg[kn])

        @pl.loop(0, NCHUNK // NBUF)
        def _(j):
            s0 = j * NBUF
            for kk in range(NBUF):
                step(s0 + kk, kk)

        for kk in range(NBUF):
            s_last = NCHUNK - NBUF + kk
            pltpu.make_async_copy(rows[kk], acc.at[dstb.at[s_last]],
                                  ss[kk]).wait()

        plsc.subcore_barrier()
        pltpu.sync_copy(acc.at[pl.ds(sid * RPS, RPS)],
                        out_hbm.at[cid].at[pl.ds(sid * RPS, RPS)])

    return k(table, src2d, dst2d, zeros)


_ROWS = 1000


def _k1_body(x_ref, wl_ref, ones_ref, wr_ref, xp_ref, xr_ref):
    xb = x_ref[...]
    xp_ref[...] = jnp.dot(xb, wl_ref[...],
                          preferred_element_type=jnp.float32) + ones_ref[...]
    xr_ref[...] = jnp.dot(xb, wr_ref[...], preferred_element_type=jnp.float32)


def _tc_project(x, wl_pad, ones_row, w1r):
    grid = N_NODES // _ROWS
    return pl.pallas_call(
        _k1_body,
        grid=(grid,),
        in_specs=[
            pl.BlockSpec((_ROWS, D_IN), lambda i: (i, 0)),
            pl.BlockSpec((D_IN, 48), lambda i: (0, 0)),
            pl.BlockSpec((1, 48), lambda i: (0, 0)),
            pl.BlockSpec((D_IN, D_HID), lambda i: (0, 0)),
        ],
        out_specs=[
            pl.BlockSpec((_ROWS, 48), lambda i: (i, 0)),
            pl.BlockSpec((_ROWS, D_HID), lambda i: (i, 0)),
        ],
        out_shape=[
            jax.ShapeDtypeStruct((N_NODES, 48), jnp.float32),
            jax.ShapeDtypeStruct((N_NODES, D_HID), jnp.float32),
        ],
    )(x, wl_pad, ones_row, w1r)


def _k2_body(agg_ref, xr_ref, b1_ref, h_ref):
    a = agg_ref[0] + agg_ref[1]
    dinv = 1.0 / jnp.clip(a[:, 32:33], 1.0, None)
    h_ref[...] = jax.nn.relu(a[:, :D_HID] * dinv + b1_ref[...] + xr_ref[...])


def _tc_hidden(agg1, xr, b1_row):
    grid = N_NODES // _ROWS
    return pl.pallas_call(
        _k2_body,
        grid=(grid,),
        in_specs=[
            pl.BlockSpec((NC, _ROWS, 48), lambda i: (0, i, 0)),
            pl.BlockSpec((_ROWS, D_HID), lambda i: (i, 0)),
            pl.BlockSpec((1, D_HID), lambda i: (0, 0)),
        ],
        out_specs=pl.BlockSpec((_ROWS, D_HID), lambda i: (i, 0)),
        out_shape=jax.ShapeDtypeStruct((N_NODES, D_HID), jnp.float32),
    )(agg1, xr, b1_row)


def _k3_body(aggh_ref, agg1_ref, h_ref, w2l_ref, b2_ref, w2r_ref, out_ref):
    deg = jnp.clip(agg1_ref[0][:, 32:33] + agg1_ref[1][:, 32:33], 1.0, None)
    m = (aggh_ref[0] + aggh_ref[1]) / deg
    out_ref[...] = (jnp.dot(m, w2l_ref[...], preferred_element_type=jnp.float32)
                    + b2_ref[...]
                    + jnp.dot(h_ref[...], w2r_ref[...],
                              preferred_element_type=jnp.float32))


def _tc_out(aggh, agg1, h, w2l, b2_row, w2r):
    grid = N_NODES // _ROWS
    return pl.pallas_call(
        _k3_body,
        grid=(grid,),
        in_specs=[
            pl.BlockSpec((NC, _ROWS, D_HID), lambda i: (0, i, 0)),
            pl.BlockSpec((NC, _ROWS, 48), lambda i: (0, i, 0)),
            pl.BlockSpec((_ROWS, D_HID), lambda i: (i, 0)),
            pl.BlockSpec((D_HID, 128), lambda i: (0, 0)),
            pl.BlockSpec((1, 128), lambda i: (0, 0)),
            pl.BlockSpec((D_HID, 128), lambda i: (0, 0)),
        ],
        out_specs=pl.BlockSpec((_ROWS, 128), lambda i: (i, 0)),
        out_shape=jax.ShapeDtypeStruct((N_NODES, 128), jnp.float32),
    )(aggh, agg1, h, w2l, b2_row, w2r)


def kernel(x, edge_index, W1_l, b1, W1_r, W2_l, b2, W2_r):
    npad = E_PAD - N_EDGES
    src = jnp.pad(edge_index[0].astype(jnp.int32), (0, npad)
                  ).reshape(NW * NCHUNK, CHUNK)
    dst = jnp.pad(edge_index[1].astype(jnp.int32), (0, npad),
                  constant_values=N_NODES).reshape(NW * NCHUNK, CHUNK)

    wl_pad = jnp.pad(W1_l, ((0, 0), (0, 16)))
    ones_row = jnp.zeros((1, 48), jnp.float32).at[0, 32].set(1.0)
    zeros48 = jnp.zeros((N_PAD, 48), jnp.float32)
    zeros32 = jnp.zeros((N_PAD, D_HID), jnp.float32)

    xp_aug, xr = _tc_project(x, wl_pad, ones_row, W1_r)
    agg1 = _sc_segment_sum(xp_aug, src, dst, zeros48, 48)
    h = _tc_hidden(agg1, xr, b1.reshape(1, D_HID))
    aggh = _sc_segment_sum(h, src, dst, zeros32, D_HID)
    out = _tc_out(aggh, agg1, h, W2_l, b2.reshape(1, 128), W2_r)
    return out


# speedup vs baseline: 9.3303x; 1.0496x over previous
import functools

import jax
import jax.numpy as jnp
from jax import lax
from jax.experimental import pallas as pl
from jax.experimental.pallas import tpu as pltpu
from jax.experimental.pallas import tpu_sc as plsc

N_NODES = 10000
N_EDGES = 320000
D_IN = 128
D_HID = 32

NC = 2
NS = 16
NW = NC * NS
CHUNK = 128
NCHUNK = 80
E_PAD = NW * NCHUNK * CHUNK
N_PAD = 10240
RPS = N_PAD // NS
NBUF = 4


def _sc_segment_sum(table, src2d, dst2d, zeros, d):
    mesh = plsc.VectorSubcoreMesh(core_axis_name="c", subcore_axis_name="s")

    @functools.partial(
        pl.kernel,
        mesh=mesh,
        compiler_params=pltpu.CompilerParams(use_tc_tiling_on_sc=False),
        out_type=jax.ShapeDtypeStruct((NC, N_PAD, d), jnp.float32),
        scratch_types=[
            pltpu.VMEM((NCHUNK, CHUNK), jnp.int32),
            pltpu.VMEM((NCHUNK, CHUNK), jnp.int32),
            [pltpu.VMEM((CHUNK, d), jnp.float32) for _ in range(NBUF)],
            pltpu.VMEM_SHARED((N_PAD, d), jnp.float32),
            pltpu.SemaphoreType.DMA,
            [pltpu.SemaphoreType.DMA for _ in range(NBUF)],
            [pltpu.SemaphoreType.DMA for _ in range(NBUF)],
        ],
    )
    def k(tab_hbm, src_hbm, dst_hbm, z_hbm, out_hbm,
          srcb, dstb, rows, acc, s_misc, sg, ss):
        cid = lax.axis_index("c")
        sid = lax.axis_index("s")
        wid = sid * NC + cid
        base = wid * NCHUNK

        pltpu.async_copy(src_hbm.at[pl.ds(base, NCHUNK)], srcb, s_misc)
        pltpu.async_copy(dst_hbm.at[pl.ds(base, NCHUNK)], dstb, s_misc)
        pltpu.async_copy(z_hbm.at[pl.ds(sid * RPS, RPS)],
                         acc.at[pl.ds(sid * RPS, RPS)], s_misc)
        pltpu.make_async_copy(src_hbm.at[pl.ds(base, NCHUNK)], srcb,
                              s_misc).wait()
        pltpu.make_async_copy(dst_hbm.at[pl.ds(base, NCHUNK)], dstb,
                              s_misc).wait()
        pltpu.make_async_copy(z_hbm.at[pl.ds(sid * RPS, RPS)],
                              acc.at[pl.ds(sid * RPS, RPS)], s_misc).wait()
        plsc.subcore_barrier()

        for kk in range(NBUF - 1):
            pltpu.async_copy(tab_hbm.at[srcb.at[kk]], rows[kk], sg[kk])

        def step(s, kk):
            pltpu.make_async_copy(tab_hbm.at[srcb.at[s]], rows[kk],
                                  sg[kk]).wait()
            pltpu.async_copy(rows[kk], acc.at[dstb.at[s]], ss[kk], add=True)
            kn = (kk + NBUF - 1) % NBUF
            sn = s + NBUF - 1

            @pl.when(sn < NCHUNK)
            def _():
                @pl.when(s >= 1)
                def _():
                    pltpu.make_async_copy(rows[kn], acc.at[dstb.at[s - 1]],
                                          ss[kn]).wait()
                pltpu.async_copy(tab_hbm.at[srcb.at[sn]], rows[kn], sg[kn])

        @pl.loop(0, NCHUNK // NBUF)
        def _(j):
            s0 = j * NBUF
            for kk in range(NBUF):
                step(s0 + kk, kk)

        for kk in range(NBUF):
            s_last = NCHUNK - NBUF + kk
            pltpu.make_async_copy(rows[kk], acc.at[dstb.at[s_last]],
                                  ss[kk]).wait()

        plsc.subcore_barrier()
        pltpu.sync_copy(acc.at[pl.ds(sid * RPS, RPS)],
                        out_hbm.at[cid].at[pl.ds(sid * RPS, RPS)])

    return k(table, src2d, dst2d, zeros)


_ROWS = 1000
DP1 = 40


def _mm_body(a_ref, w_ref, c_ref, o_ref):
    o_ref[...] = jnp.dot(a_ref[...], w_ref[...],
                         preferred_element_type=jnp.float32) + c_ref[...]


def _tc_matmul(a, w, crow):
    grid = N_NODES // _ROWS
    dk, dn = w.shape
    return pl.pallas_call(
        _mm_body,
        grid=(grid,),
        in_specs=[
            pl.BlockSpec((_ROWS, dk), lambda i: (i, 0)),
            pl.BlockSpec((dk, dn), lambda i: (0, 0)),
            pl.BlockSpec((1, dn), lambda i: (0, 0)),
        ],
        out_specs=pl.BlockSpec((_ROWS, dn), lambda i: (i, 0)),
        out_shape=jax.ShapeDtypeStruct((N_NODES, dn), jnp.float32),
    )(a, w, crow)


def _k2_body(agg_ref, xr_ref, b1_ref, h_ref):
    a = agg_ref[0] + agg_ref[1]
    dinv = 1.0 / jnp.clip(a[:, 32:33], 1.0, None)
    h_ref[...] = jax.nn.relu(a[:, :D_HID] * dinv + b1_ref[...] + xr_ref[...])


def _tc_hidden(agg1, xr, b1_row):
    grid = N_NODES // _ROWS
    return pl.pallas_call(
        _k2_body,
        grid=(grid,),
        in_specs=[
            pl.BlockSpec((NC, _ROWS, DP1), lambda i: (0, i, 0)),
            pl.BlockSpec((_ROWS, D_HID), lambda i: (i, 0)),
            pl.BlockSpec((1, D_HID), lambda i: (0, 0)),
        ],
        out_specs=pl.BlockSpec((_ROWS, D_HID), lambda i: (i, 0)),
        out_shape=jax.ShapeDtypeStruct((N_NODES, D_HID), jnp.float32),
    )(agg1, xr, b1_row)


def _k3_body(aggh_ref, agg1_ref, hr_ref, w2l_ref, out_ref):
    deg = jnp.clip(agg1_ref[0][:, 32:33] + agg1_ref[1][:, 32:33], 1.0, None)
    m = (aggh_ref[0] + aggh_ref[1]) / deg
    out_ref[...] = jnp.dot(m, w2l_ref[...],
                           preferred_element_type=jnp.float32) + hr_ref[...]


def _tc_out(aggh, agg1, hr, w2l):
    grid = N_NODES // _ROWS
    return pl.pallas_call(
        _k3_body,
        grid=(grid,),
        in_specs=[
            pl.BlockSpec((NC, _ROWS, D_HID), lambda i: (0, i, 0)),
            pl.BlockSpec((NC, _ROWS, DP1), lambda i: (0, i, 0)),
            pl.BlockSpec((_ROWS, 128), lambda i: (i, 0)),
            pl.BlockSpec((D_HID, 128), lambda i: (0, 0)),
        ],
        out_specs=pl.BlockSpec((_ROWS, 128), lambda i: (i, 0)),
        out_shape=jax.ShapeDtypeStruct((N_NODES, 128), jnp.float32),
    )(aggh, agg1, hr, w2l)


def kernel(x, edge_index, W1_l, b1, W1_r, W2_l, b2, W2_r):
    npad = E_PAD - N_EDGES
    src = jnp.pad(edge_index[0].astype(jnp.int32), (0, npad)
                  ).reshape(NW * NCHUNK, CHUNK)
    dst = jnp.pad(edge_index[1].astype(jnp.int32), (0, npad),
                  constant_values=N_NODES).reshape(NW * NCHUNK, CHUNK)

    wl_pad = jnp.pad(W1_l, ((0, 0), (0, DP1 - D_HID)))
    ones_row = jnp.zeros((1, DP1), jnp.float32).at[0, 32].set(1.0)
    zrow = jnp.zeros((1, D_HID), jnp.float32)
    zeros_p1 = jnp.zeros((N_PAD, DP1), jnp.float32)
    zeros_p2 = jnp.zeros((N_PAD, D_HID), jnp.float32)

    xp_aug = _tc_matmul(x, wl_pad, ones_row)
    agg1 = _sc_segment_sum(xp_aug, src, dst, zeros_p1, DP1)
    xr = _tc_matmul(x, W1_r, zrow)
    h = _tc_hidden(agg1, xr, b1.reshape(1, D_HID))
    aggh = _sc_segment_sum(h, src, dst, zeros_p2, D_HID)
    hr = _tc_matmul(h, W2_r, b2.reshape(1, 128))
    out = _tc_out(aggh, agg1, hr, W2_l)
    return out


# speedup vs baseline: 21.1675x; 2.2687x over previous
import functools

import jax
import jax.numpy as jnp
from jax import lax
from jax.experimental import pallas as pl
from jax.experimental.pallas import tpu as pltpu
from jax.experimental.pallas import tpu_sc as plsc

N_NODES = 10000
N_EDGES = 320000
D_IN = 128
D_HID = 32

NC = 2
NS = 16
NW = NC * NS
CHUNK = 80
NCHUNK = 125
N_PAD = 10240
RPS = N_PAD // NS
NBUF = 5


def _sc_segment_sum(table, src2d, dst2d, zeros, d):
    mesh = plsc.VectorSubcoreMesh(core_axis_name="c", subcore_axis_name="s")

    @functools.partial(
        pl.kernel,
        mesh=mesh,
        compiler_params=pltpu.CompilerParams(use_tc_tiling_on_sc=False),
        out_type=jax.ShapeDtypeStruct((NC, N_PAD, d), jnp.float32),
        scratch_types=[
            pltpu.VMEM((NCHUNK, CHUNK), jnp.int32),
            pltpu.VMEM((NCHUNK, CHUNK), jnp.int32),
            [pltpu.VMEM((CHUNK, d), jnp.float32) for _ in range(NBUF)],
            pltpu.VMEM_SHARED((N_PAD, d), jnp.float32),
            pltpu.SemaphoreType.DMA,
            [pltpu.SemaphoreType.DMA for _ in range(NBUF)],
            [pltpu.SemaphoreType.DMA for _ in range(NBUF)],
        ],
    )
    def k(tab_hbm, src_hbm, dst_hbm, z_hbm, out_hbm,
          srcb, dstb, rows, acc, s_misc, sg, ss):
        cid = lax.axis_index("c")
        sid = lax.axis_index("s")
        wid = sid * NC + cid
        base = wid * NCHUNK

        pltpu.async_copy(src_hbm.at[pl.ds(base, NCHUNK)], srcb, s_misc)
        pltpu.async_copy(dst_hbm.at[pl.ds(base, NCHUNK)], dstb, s_misc)
        pltpu.async_copy(z_hbm.at[pl.ds(sid * RPS, RPS)],
                         acc.at[pl.ds(sid * RPS, RPS)], s_misc)
        pltpu.make_async_copy(src_hbm.at[pl.ds(base, NCHUNK)], srcb,
                              s_misc).wait()
        pltpu.make_async_copy(dst_hbm.at[pl.ds(base, NCHUNK)], dstb,
                              s_misc).wait()
        pltpu.make_async_copy(z_hbm.at[pl.ds(sid * RPS, RPS)],
                              acc.at[pl.ds(sid * RPS, RPS)], s_misc).wait()
        plsc.subcore_barrier()

        for kk in range(NBUF - 1):
            pltpu.async_copy(tab_hbm.at[srcb.at[kk]], rows[kk], sg[kk])

        def step(s, kk):
            pltpu.make_async_copy(tab_hbm.at[srcb.at[s]], rows[kk],
                                  sg[kk]).wait()
            pltpu.async_copy(rows[kk], acc.at[dstb.at[s]], ss[kk], add=True)
            kn = (kk + NBUF - 1) % NBUF
            sn = s + NBUF - 1

            @pl.when(sn < NCHUNK)
            def _():
                @pl.when(s >= 1)
                def _():
                    pltpu.make_async_copy(rows[kn], acc.at[dstb.at[s - 1]],
                                          ss[kn]).wait()
                pltpu.async_copy(tab_hbm.at[srcb.at[sn]], rows[kn], sg[kn])

        @pl.loop(0, NCHUNK // NBUF)
        def _(j):
            s0 = j * NBUF
            for kk in range(NBUF):
                step(s0 + kk, kk)

        for kk in range(NBUF):
            s_last = NCHUNK - NBUF + kk
            pltpu.make_async_copy(rows[kk], acc.at[dstb.at[s_last]],
                                  ss[kk]).wait()

        plsc.subcore_barrier()
        pltpu.sync_copy(acc.at[pl.ds(sid * RPS, RPS)],
                        out_hbm.at[cid].at[pl.ds(sid * RPS, RPS)])

    return k(table, src2d, dst2d, zeros)


_ROWS = 1000
DP1 = 40


def _mm_body(a_ref, w_ref, c_ref, o_ref):
    o_ref[...] = jnp.dot(a_ref[...], w_ref[...],
                         preferred_element_type=jnp.float32) + c_ref[...]


def _tc_matmul(a, w, crow):
    grid = N_NODES // _ROWS
    dk, dn = w.shape
    return pl.pallas_call(
        _mm_body,
        grid=(grid,),
        in_specs=[
            pl.BlockSpec((_ROWS, dk), lambda i: (i, 0)),
            pl.BlockSpec((dk, dn), lambda i: (0, 0)),
            pl.BlockSpec((1, dn), lambda i: (0, 0)),
        ],
        out_specs=pl.BlockSpec((_ROWS, dn), lambda i: (i, 0)),
        out_shape=jax.ShapeDtypeStruct((N_NODES, dn), jnp.float32),
    )(a, w, crow)


def _k2_body(agg_ref, xr_ref, b1_ref, h_ref):
    a = agg_ref[0] + agg_ref[1]
    dinv = 1.0 / jnp.clip(a[:, 32:33], 1.0, None)
    h_ref[...] = jax.nn.relu(a[:, :D_HID] * dinv + b1_ref[...] + xr_ref[...])


def _tc_hidden(agg1, xr, b1_row):
    grid = N_NODES // _ROWS
    return pl.pallas_call(
        _k2_body,
        grid=(grid,),
        in_specs=[
            pl.BlockSpec((NC, _ROWS, DP1), lambda i: (0, i, 0)),
            pl.BlockSpec((_ROWS, D_HID), lambda i: (i, 0)),
            pl.BlockSpec((1, D_HID), lambda i: (0, 0)),
        ],
        out_specs=pl.BlockSpec((_ROWS, D_HID), lambda i: (i, 0)),
        out_shape=jax.ShapeDtypeStruct((N_NODES, D_HID), jnp.float32),
    )(agg1, xr, b1_row)


def _k3_body(aggh_ref, agg1_ref, hr_ref, w2l_ref, out_ref):
    deg = jnp.clip(agg1_ref[0][:, 32:33] + agg1_ref[1][:, 32:33], 1.0, None)
    m = (aggh_ref[0] + aggh_ref[1]) / deg
    out_ref[...] = jnp.dot(m, w2l_ref[...],
                           preferred_element_type=jnp.float32) + hr_ref[...]


def _tc_out(aggh, agg1, hr, w2l):
    grid = N_NODES // _ROWS
    return pl.pallas_call(
        _k3_body,
        grid=(grid,),
        in_specs=[
            pl.BlockSpec((NC, _ROWS, D_HID), lambda i: (0, i, 0)),
            pl.BlockSpec((NC, _ROWS, DP1), lambda i: (0, i, 0)),
            pl.BlockSpec((_ROWS, 128), lambda i: (i, 0)),
            pl.BlockSpec((D_HID, 128), lambda i: (0, 0)),
        ],
        out_specs=pl.BlockSpec((_ROWS, 128), lambda i: (i, 0)),
        out_shape=jax.ShapeDtypeStruct((N_NODES, 128), jnp.float32),
    )(aggh, agg1, hr, w2l)


def kernel(x, edge_index, W1_l, b1, W1_r, W2_l, b2, W2_r):
    src = edge_index[0].astype(jnp.int32).reshape(NW * NCHUNK, CHUNK)
    dst = edge_index[1].astype(jnp.int32).reshape(NW * NCHUNK, CHUNK)

    wl_pad = jnp.pad(W1_l, ((0, 0), (0, DP1 - D_HID)))
    ones_row = jnp.zeros((1, DP1), jnp.float32).at[0, 32].set(1.0)
    zrow = jnp.zeros((1, D_HID), jnp.float32)
    zeros_p1 = jnp.zeros((N_PAD, DP1), jnp.float32)
    zeros_p2 = jnp.zeros((N_PAD, D_HID), jnp.float32)

    xp_aug = _tc_matmul(x, wl_pad, ones_row)
    agg1 = _sc_segment_sum(xp_aug, src, dst, zeros_p1, DP1)
    xr = _tc_matmul(x, W1_r, zrow)
    h = _tc_hidden(agg1, xr, b1.reshape(1, D_HID))
    aggh = _sc_segment_sum(h, src, dst, zeros_p2, D_HID)
    hr = _tc_matmul(h, W2_r, b2.reshape(1, 128))
    out = _tc_out(aggh, agg1, hr, W2_l)
    return out


# speedup vs baseline: 22.2198x; 1.0497x over previous
import functools

import jax
import jax.numpy as jnp
from jax import lax
from jax.experimental import pallas as pl
from jax.experimental.pallas import tpu as pltpu
from jax.experimental.pallas import tpu_sc as plsc

N_NODES = 10000
N_EDGES = 320000
D_IN = 128
D_HID = 32

NC = 2
NS = 16
NW = NC * NS
CHUNK = 80
NCHUNK = 125
N_PAD = 10240
RPS = N_PAD // NS
NBUF = 5


def _sc_segment_sum(table, src2d, dst2d, zeros, d):
    mesh = plsc.VectorSubcoreMesh(core_axis_name="c", subcore_axis_name="s")

    @functools.partial(
        pl.kernel,
        mesh=mesh,
        compiler_params=pltpu.CompilerParams(use_tc_tiling_on_sc=False),
        out_type=jax.ShapeDtypeStruct((NC, N_PAD, d), jnp.float32),
        scratch_types=[
            pltpu.VMEM((NCHUNK, CHUNK), jnp.int32),
            pltpu.VMEM((NCHUNK, CHUNK), jnp.int32),
            [pltpu.VMEM((CHUNK, d), jnp.float32) for _ in range(NBUF)],
            pltpu.VMEM_SHARED((N_PAD, d), jnp.float32),
            pltpu.SemaphoreType.DMA,
            [pltpu.SemaphoreType.DMA for _ in range(NBUF)],
            [pltpu.SemaphoreType.DMA for _ in range(NBUF)],
        ],
    )
    def k(tab_hbm, src_hbm, dst_hbm, z_hbm, out_hbm,
          srcb, dstb, rows, acc, s_misc, sg, ss):
        cid = lax.axis_index("c")
        sid = lax.axis_index("s")
        wid = sid * NC + cid
        base = wid * NCHUNK

        pltpu.async_copy(src_hbm.at[pl.ds(base, NCHUNK)], srcb, s_misc)
        pltpu.async_copy(dst_hbm.at[pl.ds(base, NCHUNK)], dstb, s_misc)
        pltpu.async_copy(z_hbm.at[pl.ds(sid * RPS, RPS)],
                         acc.at[pl.ds(sid * RPS, RPS)], s_misc)
        pltpu.make_async_copy(src_hbm.at[pl.ds(base, NCHUNK)], srcb,
                              s_misc).wait()
        pltpu.make_async_copy(dst_hbm.at[pl.ds(base, NCHUNK)], dstb,
                              s_misc).wait()
        pltpu.make_async_copy(z_hbm.at[pl.ds(sid * RPS, RPS)],
                              acc.at[pl.ds(sid * RPS, RPS)], s_misc).wait()
        plsc.subcore_barrier()

        for kk in range(NBUF - 1):
            pltpu.async_copy(tab_hbm.at[srcb.at[kk]], rows[kk], sg[kk])

        def step(s, kk):
            pltpu.make_async_copy(tab_hbm.at[srcb.at[s]], rows[kk],
                                  sg[kk]).wait()
            pltpu.async_copy(rows[kk], acc.at[dstb.at[s]], ss[kk], add=True)
            kn = (kk + NBUF - 1) % NBUF
            sn = s + NBUF - 1

            @pl.when(sn < NCHUNK)
            def _():
                @pl.when(s >= 1)
                def _():
                    pltpu.make_async_copy(rows[kn], acc.at[dstb.at[s - 1]],
                                          ss[kn]).wait()
                pltpu.async_copy(tab_hbm.at[srcb.at[sn]], rows[kn], sg[kn])

        @pl.loop(0, NCHUNK // NBUF)
        def _(j):
            s0 = j * NBUF
            for kk in range(NBUF):
                step(s0 + kk, kk)

        for kk in range(NBUF):
            s_last = NCHUNK - NBUF + kk
            pltpu.make_async_copy(rows[kk], acc.at[dstb.at[s_last]],
                                  ss[kk]).wait()

        plsc.subcore_barrier()
        pltpu.sync_copy(acc.at[pl.ds(sid * RPS, RPS)],
                        out_hbm.at[cid].at[pl.ds(sid * RPS, RPS)])

    return k(table, src2d, dst2d, zeros)


_ROWS = N_NODES
DP1 = 40


def _mm_body(a_ref, w_ref, c_ref, o_ref):
    o_ref[...] = jnp.dot(a_ref[...], w_ref[...],
                         preferred_element_type=jnp.float32) + c_ref[...]


def _tc_matmul(a, w, crow):
    grid = N_NODES // _ROWS
    dk, dn = w.shape
    return pl.pallas_call(
        _mm_body,
        grid=(grid,),
        in_specs=[
            pl.BlockSpec((_ROWS, dk), lambda i: (i, 0)),
            pl.BlockSpec((dk, dn), lambda i: (0, 0)),
            pl.BlockSpec((1, dn), lambda i: (0, 0)),
        ],
        out_specs=pl.BlockSpec((_ROWS, dn), lambda i: (i, 0)),
        out_shape=jax.ShapeDtypeStruct((N_NODES, dn), jnp.float32),
    )(a, w, crow)


def _k2_body(agg_ref, xr_ref, b1_ref, h_ref):
    a = agg_ref[0] + agg_ref[1]
    dinv = 1.0 / jnp.clip(a[:, 32:33], 1.0, None)
    h_ref[...] = jax.nn.relu(a[:, :D_HID] * dinv + b1_ref[...] + xr_ref[...])


def _tc_hidden(agg1, xr, b1_row):
    grid = N_NODES // _ROWS
    return pl.pallas_call(
        _k2_body,
        grid=(grid,),
        in_specs=[
            pl.BlockSpec((NC, _ROWS, DP1), lambda i: (0, i, 0)),
            pl.BlockSpec((_ROWS, D_HID), lambda i: (i, 0)),
            pl.BlockSpec((1, D_HID), lambda i: (0, 0)),
        ],
        out_specs=pl.BlockSpec((_ROWS, D_HID), lambda i: (i, 0)),
        out_shape=jax.ShapeDtypeStruct((N_NODES, D_HID), jnp.float32),
    )(agg1, xr, b1_row)


def _k3_body(aggh_ref, agg1_ref, hr_ref, w2l_ref, out_ref):
    deg = jnp.clip(agg1_ref[0][:, 32:33] + agg1_ref[1][:, 32:33], 1.0, None)
    m = (aggh_ref[0] + aggh_ref[1]) / deg
    out_ref[...] = jnp.dot(m, w2l_ref[...],
                           preferred_element_type=jnp.float32) + hr_ref[...]


def _tc_out(aggh, agg1, hr, w2l):
    grid = N_NODES // _ROWS
    return pl.pallas_call(
        _k3_body,
        grid=(grid,),
        in_specs=[
            pl.BlockSpec((NC, _ROWS, D_HID), lambda i: (0, i, 0)),
            pl.BlockSpec((NC, _ROWS, DP1), lambda i: (0, i, 0)),
            pl.BlockSpec((_ROWS, 128), lambda i: (i, 0)),
            pl.BlockSpec((D_HID, 128), lambda i: (0, 0)),
        ],
        out_specs=pl.BlockSpec((_ROWS, 128), lambda i: (i, 0)),
        out_shape=jax.ShapeDtypeStruct((N_NODES, 128), jnp.float32),
    )(aggh, agg1, hr, w2l)


def kernel(x, edge_index, W1_l, b1, W1_r, W2_l, b2, W2_r):
    src = edge_index[0].astype(jnp.int32).reshape(NW * NCHUNK, CHUNK)
    dst = edge_index[1].astype(jnp.int32).reshape(NW * NCHUNK, CHUNK)

    wl_pad = jnp.pad(W1_l, ((0, 0), (0, DP1 - D_HID)))
    ones_row = jnp.zeros((1, DP1), jnp.float32).at[0, 32].set(1.0)
    zrow = jnp.zeros((1, D_HID), jnp.float32)
    zeros_p1 = jnp.zeros((N_PAD, DP1), jnp.float32)
    zeros_p2 = jnp.zeros((N_PAD, D_HID), jnp.float32)

    xp_aug = _tc_matmul(x, wl_pad, ones_row)
    agg1 = _sc_segment_sum(xp_aug, src, dst, zeros_p1, DP1)
    xr = _tc_matmul(x, W1_r, zrow)
    h = _tc_hidden(agg1, xr, b1.reshape(1, D_HID))
    aggh = _sc_segment_sum(h, src, dst, zeros_p2, D_HID)
    hr = _tc_matmul(h, W2_r, b2.reshape(1, 128))
    out = _tc_out(aggh, agg1, hr, W2_l)
    return out


# speedup vs baseline: 22.2320x; 1.0006x over previous
import functools

import jax
import jax.numpy as jnp
from jax import lax
from jax.experimental import pallas as pl
from jax.experimental.pallas import tpu as pltpu
from jax.experimental.pallas import tpu_sc as plsc

N_NODES = 10000
N_EDGES = 320000
D_IN = 128
D_HID = 32

NC = 2
NS = 16
NW = NC * NS
CHUNK = 80
NCHUNK = 125
N_PAD = 10240
RPS = N_PAD // NS
NBUF = 5


def _sc_segment_sum(table, src1d, dst1d, zeros, d):
    mesh = plsc.VectorSubcoreMesh(core_axis_name="c", subcore_axis_name="s")

    @functools.partial(
        pl.kernel,
        mesh=mesh,
        compiler_params=pltpu.CompilerParams(use_tc_tiling_on_sc=False),
        out_type=jax.ShapeDtypeStruct((NC, N_PAD, d), jnp.float32),
        scratch_types=[
            pltpu.VMEM((NCHUNK * CHUNK,), jnp.int32),
            pltpu.VMEM((NCHUNK * CHUNK,), jnp.int32),
            [pltpu.VMEM((CHUNK, d), jnp.float32) for _ in range(NBUF)],
            pltpu.VMEM_SHARED((N_PAD, d), jnp.float32),
            pltpu.SemaphoreType.DMA,
            [pltpu.SemaphoreType.DMA for _ in range(NBUF)],
            [pltpu.SemaphoreType.DMA for _ in range(NBUF)],
        ],
    )
    def k(tab_hbm, src_hbm, dst_hbm, z_hbm, out_hbm,
          srcb, dstb, rows, acc, s_misc, sg, ss):
        cid = lax.axis_index("c")
        sid = lax.axis_index("s")
        wid = sid * NC + cid
        base = wid * NCHUNK * CHUNK

        pltpu.async_copy(src_hbm.at[pl.ds(base, NCHUNK * CHUNK)], srcb, s_misc)
        pltpu.async_copy(dst_hbm.at[pl.ds(base, NCHUNK * CHUNK)], dstb, s_misc)
        pltpu.async_copy(z_hbm.at[pl.ds(sid * RPS, RPS)],
                         acc.at[pl.ds(sid * RPS, RPS)], s_misc)
        pltpu.make_async_copy(src_hbm.at[pl.ds(base, NCHUNK * CHUNK)], srcb,
                              s_misc).wait()
        pltpu.make_async_copy(dst_hbm.at[pl.ds(base, NCHUNK * CHUNK)], dstb,
                              s_misc).wait()
        pltpu.make_async_copy(z_hbm.at[pl.ds(sid * RPS, RPS)],
                              acc.at[pl.ds(sid * RPS, RPS)], s_misc).wait()
        plsc.subcore_barrier()

        for kk in range(NBUF - 1):
            pltpu.async_copy(tab_hbm.at[srcb.at[pl.ds(kk * CHUNK, CHUNK)]], rows[kk], sg[kk])

        def step(s, kk):
            pltpu.make_async_copy(tab_hbm.at[srcb.at[pl.ds(s * CHUNK, CHUNK)]], rows[kk],
                                  sg[kk]).wait()
            pltpu.async_copy(rows[kk], acc.at[dstb.at[pl.ds(s * CHUNK, CHUNK)]], ss[kk], add=True)
            kn = (kk + NBUF - 1) % NBUF
            sn = s + NBUF - 1

            @pl.when(sn < NCHUNK)
            def _():
                @pl.when(s >= 1)
                def _():
                    pltpu.make_async_copy(rows[kn], acc.at[dstb.at[pl.ds((s - 1) * CHUNK, CHUNK)]],
                                          ss[kn]).wait()
                pltpu.async_copy(tab_hbm.at[srcb.at[pl.ds(sn * CHUNK, CHUNK)]], rows[kn], sg[kn])

        @pl.loop(0, NCHUNK // NBUF)
        def _(j):
            s0 = j * NBUF
            for kk in range(NBUF):
                step(s0 + kk, kk)

        for kk in range(NBUF):
            s_last = NCHUNK - NBUF + kk
            pltpu.make_async_copy(rows[kk], acc.at[dstb.at[pl.ds(s_last * CHUNK, CHUNK)]],
                                  ss[kk]).wait()

        plsc.subcore_barrier()
        pltpu.sync_copy(acc.at[pl.ds(sid * RPS, RPS)],
                        out_hbm.at[cid].at[pl.ds(sid * RPS, RPS)])

    return k(table, src1d, dst1d, zeros)


_ROWS = N_NODES
DP1 = 40


def _mm_body(a_ref, w_ref, c_ref, o_ref):
    o_ref[...] = jnp.dot(a_ref[...], w_ref[...],
                         preferred_element_type=jnp.float32) + c_ref[...]


def _tc_matmul(a, w, crow):
    grid = N_NODES // _ROWS
    dk, dn = w.shape
    return pl.pallas_call(
        _mm_body,
        grid=(grid,),
        in_specs=[
            pl.BlockSpec((_ROWS, dk), lambda i: (i, 0)),
            pl.BlockSpec((dk, dn), lambda i: (0, 0)),
            pl.BlockSpec((1, dn), lambda i: (0, 0)),
        ],
        out_specs=pl.BlockSpec((_ROWS, dn), lambda i: (i, 0)),
        out_shape=jax.ShapeDtypeStruct((N_NODES, dn), jnp.float32),
    )(a, w, crow)


def _k2_body(agg_ref, xr_ref, b1_ref, h_ref):
    a = agg_ref[0] + agg_ref[1]
    dinv = 1.0 / jnp.clip(a[:, 32:33], 1.0, None)
    h_ref[...] = jax.nn.relu(a[:, :D_HID] * dinv + b1_ref[...] + xr_ref[...])


def _tc_hidden(agg1, xr, b1_row):
    grid = N_NODES // _ROWS
    return pl.pallas_call(
        _k2_body,
        grid=(grid,),
        in_specs=[
            pl.BlockSpec((NC, _ROWS, DP1), lambda i: (0, i, 0)),
            pl.BlockSpec((_ROWS, D_HID), lambda i: (i, 0)),
            pl.BlockSpec((1, D_HID), lambda i: (0, 0)),
        ],
        out_specs=pl.BlockSpec((_ROWS, D_HID), lambda i: (i, 0)),
        out_shape=jax.ShapeDtypeStruct((N_NODES, D_HID), jnp.float32),
    )(agg1, xr, b1_row)


def _k3_body(aggh_ref, agg1_ref, hr_ref, w2l_ref, out_ref):
    deg = jnp.clip(agg1_ref[0][:, 32:33] + agg1_ref[1][:, 32:33], 1.0, None)
    m = (aggh_ref[0] + aggh_ref[1]) / deg
    out_ref[...] = jnp.dot(m, w2l_ref[...],
                           preferred_element_type=jnp.float32) + hr_ref[...]


def _tc_out(aggh, agg1, hr, w2l):
    grid = N_NODES // _ROWS
    return pl.pallas_call(
        _k3_body,
        grid=(grid,),
        in_specs=[
            pl.BlockSpec((NC, _ROWS, D_HID), lambda i: (0, i, 0)),
            pl.BlockSpec((NC, _ROWS, DP1), lambda i: (0, i, 0)),
            pl.BlockSpec((_ROWS, 128), lambda i: (i, 0)),
            pl.BlockSpec((D_HID, 128), lambda i: (0, 0)),
        ],
        out_specs=pl.BlockSpec((_ROWS, 128), lambda i: (i, 0)),
        out_shape=jax.ShapeDtypeStruct((N_NODES, 128), jnp.float32),
    )(aggh, agg1, hr, w2l)


def kernel(x, edge_index, W1_l, b1, W1_r, W2_l, b2, W2_r):
    src = edge_index[0].astype(jnp.int32)
    dst = edge_index[1].astype(jnp.int32)

    wl_pad = jnp.pad(W1_l, ((0, 0), (0, DP1 - D_HID)))
    ones_row = jnp.zeros((1, DP1), jnp.float32).at[0, 32].set(1.0)
    zrow = jnp.zeros((1, D_HID), jnp.float32)
    zeros_p1 = jnp.zeros((N_PAD, DP1), jnp.float32)
    zeros_p2 = jnp.zeros((N_PAD, D_HID), jnp.float32)

    xp_aug = _tc_matmul(x, wl_pad, ones_row)
    agg1 = _sc_segment_sum(xp_aug, src, dst, zeros_p1, DP1)
    xr = _tc_matmul(x, W1_r, zrow)
    h = _tc_hidden(agg1, xr, b1.reshape(1, D_HID))
    aggh = _sc_segment_sum(h, src, dst, zeros_p2, D_HID)
    hr = _tc_matmul(h, W2_r, b2.reshape(1, 128))
    out = _tc_out(aggh, agg1, hr, W2_l)
    return out


# speedup vs baseline: 24.4166x; 1.0983x over previous
import functools

import jax
import jax.numpy as jnp
from jax import lax
from jax.experimental import pallas as pl
from jax.experimental.pallas import tpu as pltpu
from jax.experimental.pallas import tpu_sc as plsc

N_NODES = 10000
N_EDGES = 320000
D_IN = 128
D_HID = 32

NC = 2
NS = 16
NW = NC * NS
CHUNK = 80
NCHUNK = 125
N_PAD = 10240
RPS = N_PAD // NS
NBUF = 5


def _sc_segment_sum(table, src1d, dst1d, zeros, d):
    mesh = plsc.VectorSubcoreMesh(core_axis_name="c", subcore_axis_name="s")

    @functools.partial(
        pl.kernel,
        mesh=mesh,
        compiler_params=pltpu.CompilerParams(use_tc_tiling_on_sc=False),
        out_type=jax.ShapeDtypeStruct((NC, N_PAD, 128), jnp.float32),
        scratch_types=[
            pltpu.VMEM((NCHUNK * CHUNK,), jnp.int32),
            pltpu.VMEM((NCHUNK * CHUNK,), jnp.int32),
            [pltpu.VMEM((CHUNK, d), jnp.float32) for _ in range(NBUF)],
            pltpu.VMEM_SHARED((N_PAD, d), jnp.float32),
            pltpu.SemaphoreType.DMA,
            [pltpu.SemaphoreType.DMA for _ in range(NBUF)],
            [pltpu.SemaphoreType.DMA for _ in range(NBUF)],
        ],
    )
    def k(tab_hbm, src_hbm, dst_hbm, z_hbm, out_hbm,
          srcb, dstb, rows, acc, s_misc, sg, ss):
        cid = lax.axis_index("c")
        sid = lax.axis_index("s")
        wid = sid * NC + cid
        base = wid * NCHUNK * CHUNK

        pltpu.async_copy(src_hbm.at[pl.ds(base, NCHUNK * CHUNK)], srcb, s_misc)
        pltpu.async_copy(dst_hbm.at[pl.ds(base, NCHUNK * CHUNK)], dstb, s_misc)
        pltpu.async_copy(z_hbm.at[pl.ds(sid * RPS, RPS)],
                         acc.at[pl.ds(sid * RPS, RPS)], s_misc)
        pltpu.make_async_copy(src_hbm.at[pl.ds(base, NCHUNK * CHUNK)], srcb,
                              s_misc).wait()
        pltpu.make_async_copy(dst_hbm.at[pl.ds(base, NCHUNK * CHUNK)], dstb,
                              s_misc).wait()
        pltpu.make_async_copy(z_hbm.at[pl.ds(sid * RPS, RPS)],
                              acc.at[pl.ds(sid * RPS, RPS)], s_misc).wait()
        plsc.subcore_barrier()

        for kk in range(NBUF - 1):
            pltpu.async_copy(tab_hbm.at[srcb.at[pl.ds(kk * CHUNK, CHUNK)]], rows[kk], sg[kk])

        def step(s, kk):
            pltpu.make_async_copy(tab_hbm.at[srcb.at[pl.ds(s * CHUNK, CHUNK)]], rows[kk],
                                  sg[kk]).wait()
            pltpu.async_copy(rows[kk], acc.at[dstb.at[pl.ds(s * CHUNK, CHUNK)]], ss[kk], add=True)
            kn = (kk + NBUF - 1) % NBUF
            sn = s + NBUF - 1

            @pl.when(sn < NCHUNK)
            def _():
                @pl.when(s >= 1)
                def _():
                    pltpu.make_async_copy(rows[kn], acc.at[dstb.at[pl.ds((s - 1) * CHUNK, CHUNK)]],
                                          ss[kn]).wait()
                pltpu.async_copy(tab_hbm.at[srcb.at[pl.ds(sn * CHUNK, CHUNK)]], rows[kn], sg[kn])

        @pl.loop(0, NCHUNK // NBUF)
        def _(j):
            s0 = j * NBUF
            for kk in range(NBUF):
                step(s0 + kk, kk)

        for kk in range(NBUF):
            s_last = NCHUNK - NBUF + kk
            pltpu.make_async_copy(rows[kk], acc.at[dstb.at[pl.ds(s_last * CHUNK, CHUNK)]],
                                  ss[kk]).wait()

        plsc.subcore_barrier()
        pltpu.sync_copy(acc.at[pl.ds(sid * RPS, RPS)],
                        out_hbm.at[cid].at[pl.ds(sid * RPS, RPS), pl.ds(0, d)])

    return k(table, src1d, dst1d, zeros)


_ROWS = N_NODES
DP1 = 40


def _mm_body(a_ref, w_ref, c_ref, o_ref):
    o_ref[...] = jnp.dot(a_ref[...], w_ref[...],
                         preferred_element_type=jnp.float32) + c_ref[...]


def _tc_matmul(a, w, crow):
    grid = N_NODES // _ROWS
    dk, dn = w.shape
    return pl.pallas_call(
        _mm_body,
        grid=(grid,),
        in_specs=[
            pl.BlockSpec((_ROWS, dk), lambda i: (i, 0)),
            pl.BlockSpec((dk, dn), lambda i: (0, 0)),
            pl.BlockSpec((1, dn), lambda i: (0, 0)),
        ],
        out_specs=pl.BlockSpec((_ROWS, dn), lambda i: (i, 0)),
        out_shape=jax.ShapeDtypeStruct((N_NODES, dn), jnp.float32),
    )(a, w, crow)


def _k2_body(agg_ref, xr_ref, b1_ref, h_ref):
    a = agg_ref[0] + agg_ref[1]
    dinv = 1.0 / jnp.clip(a[:, 32:33], 1.0, None)
    h_ref[...] = jax.nn.relu(a[:, :D_HID] * dinv + b1_ref[...] + xr_ref[...])


def _tc_hidden(agg1, xr, b1_row):
    grid = N_NODES // _ROWS
    return pl.pallas_call(
        _k2_body,
        grid=(grid,),
        in_specs=[
            pl.BlockSpec((NC, _ROWS, 128), lambda i: (0, i, 0)),
            pl.BlockSpec((_ROWS, D_HID), lambda i: (i, 0)),
            pl.BlockSpec((1, D_HID), lambda i: (0, 0)),
        ],
        out_specs=pl.BlockSpec((_ROWS, D_HID), lambda i: (i, 0)),
        out_shape=jax.ShapeDtypeStruct((N_NODES, D_HID), jnp.float32),
    )(agg1, xr, b1_row)


def _k3_body(aggh_ref, agg1_ref, hr_ref, w2l_ref, out_ref):
    deg = jnp.clip(agg1_ref[0][:, 32:33] + agg1_ref[1][:, 32:33], 1.0, None)
    m = (aggh_ref[0][:, :D_HID] + aggh_ref[1][:, :D_HID]) / deg
    out_ref[...] = jnp.dot(m, w2l_ref[...],
                           preferred_element_type=jnp.float32) + hr_ref[...]


def _tc_out(aggh, agg1, hr, w2l):
    grid = N_NODES // _ROWS
    return pl.pallas_call(
        _k3_body,
        grid=(grid,),
        in_specs=[
            pl.BlockSpec((NC, _ROWS, 128), lambda i: (0, i, 0)),
            pl.BlockSpec((NC, _ROWS, 128), lambda i: (0, i, 0)),
            pl.BlockSpec((_ROWS, 128), lambda i: (i, 0)),
            pl.BlockSpec((D_HID, 128), lambda i: (0, 0)),
        ],
        out_specs=pl.BlockSpec((_ROWS, 128), lambda i: (i, 0)),
        out_shape=jax.ShapeDtypeStruct((N_NODES, 128), jnp.float32),
    )(aggh, agg1, hr, w2l)


def kernel(x, edge_index, W1_l, b1, W1_r, W2_l, b2, W2_r):
    src = edge_index[0].astype(jnp.int32)
    dst = edge_index[1].astype(jnp.int32)

    wl_pad = jnp.pad(W1_l, ((0, 0), (0, DP1 - D_HID)))
    ones_row = jnp.zeros((1, DP1), jnp.float32).at[0, 32].set(1.0)
    zrow = jnp.zeros((1, D_HID), jnp.float32)
    zeros_p1 = jnp.zeros((N_PAD, DP1), jnp.float32)
    zeros_p2 = jnp.zeros((N_PAD, D_HID), jnp.float32)

    xp_aug = _tc_matmul(x, wl_pad, ones_row)
    agg1 = _sc_segment_sum(xp_aug, src, dst, zeros_p1, DP1)
    xr = _tc_matmul(x, W1_r, zrow)
    h = _tc_hidden(agg1, xr, b1.reshape(1, D_HID))
    aggh = _sc_segment_sum(h, src, dst, zeros_p2, D_HID)
    hr = _tc_matmul(h, W2_r, b2.reshape(1, 128))
    out = _tc_out(aggh, agg1, hr, W2_l)
    return out


# speedup vs baseline: 26.7014x; 1.0936x over previous
import functools

import jax
import jax.numpy as jnp
from jax import lax
from jax.experimental import pallas as pl
from jax.experimental.pallas import tpu as pltpu
from jax.experimental.pallas import tpu_sc as plsc

N_NODES = 10000
N_EDGES = 320000
D_IN = 128
D_HID = 32

NC = 2
NS = 16
NW = NC * NS
CHUNK = 80
NCHUNK = 125
N_PAD = 10240
RPS = N_PAD // NS
NBUF = 5


def _sc_segment_sum(table, src1d, dst1d, zeros, d):
    mesh = plsc.VectorSubcoreMesh(core_axis_name="c", subcore_axis_name="s")

    @functools.partial(
        pl.kernel,
        mesh=mesh,
        compiler_params=pltpu.CompilerParams(use_tc_tiling_on_sc=False),
        out_type=jax.ShapeDtypeStruct((NC, N_PAD, 128), jnp.float32),
        scratch_types=[
            pltpu.VMEM((NCHUNK * CHUNK,), jnp.int32),
            pltpu.VMEM((NCHUNK * CHUNK,), jnp.int32),
            [pltpu.VMEM((CHUNK, d), jnp.float32) for _ in range(NBUF)],
            pltpu.VMEM_SHARED((N_PAD, d), jnp.float32),
            pltpu.SemaphoreType.DMA,
            [pltpu.SemaphoreType.DMA for _ in range(NBUF)],
            [pltpu.SemaphoreType.DMA for _ in range(NBUF)],
        ],
    )
    def k(tab_hbm, src_hbm, dst_hbm, z_hbm, out_hbm,
          srcb, dstb, rows, acc, s_misc, sg, ss):
        cid = lax.axis_index("c")
        sid = lax.axis_index("s")
        wid = sid * NC + cid
        base = wid * NCHUNK * CHUNK

        pltpu.async_copy(src_hbm.at[pl.ds(base, NCHUNK * CHUNK)], srcb, s_misc)
        pltpu.async_copy(dst_hbm.at[pl.ds(base, NCHUNK * CHUNK)], dstb, s_misc)
        pltpu.async_copy(z_hbm.at[pl.ds(sid * RPS, RPS)],
                         acc.at[pl.ds(sid * RPS, RPS)], s_misc)
        pltpu.make_async_copy(src_hbm.at[pl.ds(base, NCHUNK * CHUNK)], srcb,
                              s_misc).wait()
        pltpu.make_async_copy(dst_hbm.at[pl.ds(base, NCHUNK * CHUNK)], dstb,
                              s_misc).wait()
        pltpu.make_async_copy(z_hbm.at[pl.ds(sid * RPS, RPS)],
                              acc.at[pl.ds(sid * RPS, RPS)], s_misc).wait()
        plsc.subcore_barrier()

        for kk in range(NBUF - 1):
            pltpu.async_copy(tab_hbm.at[srcb.at[pl.ds(kk * CHUNK, CHUNK)]], rows[kk], sg[kk])

        def step(s, kk):
            pltpu.make_async_copy(tab_hbm.at[srcb.at[pl.ds(s * CHUNK, CHUNK)]], rows[kk],
                                  sg[kk]).wait()
            pltpu.async_copy(rows[kk], acc.at[dstb.at[pl.ds(s * CHUNK, CHUNK)]], ss[kk], add=True)
            kn = (kk + NBUF - 1) % NBUF
            sn = s + NBUF - 1

            @pl.when(sn < NCHUNK)
            def _():
                @pl.when(s >= 1)
                def _():
                    pltpu.make_async_copy(rows[kn], acc.at[dstb.at[pl.ds((s - 1) * CHUNK, CHUNK)]],
                                          ss[kn]).wait()
                pltpu.async_copy(tab_hbm.at[srcb.at[pl.ds(sn * CHUNK, CHUNK)]], rows[kn], sg[kn])

        @pl.loop(0, NCHUNK // NBUF)
        def _(j):
            s0 = j * NBUF
            for kk in range(NBUF):
                step(s0 + kk, kk)

        for kk in range(NBUF):
            s_last = NCHUNK - NBUF + kk
            pltpu.make_async_copy(rows[kk], acc.at[dstb.at[pl.ds(s_last * CHUNK, CHUNK)]],
                                  ss[kk]).wait()

        plsc.subcore_barrier()
        pltpu.sync_copy(acc.at[pl.ds(sid * RPS, RPS)],
                        out_hbm.at[cid].at[pl.ds(sid * RPS, RPS), pl.ds(0, d)])

    return k(table, src1d, dst1d, zeros)


_ROWS = N_NODES


def _esplit_body(e_ref, s_ref, d_ref):
    s_ref[...] = e_ref[0]
    d_ref[...] = e_ref[1]


def _tc_edge_split(e):
    return pl.pallas_call(
        _esplit_body,
        in_specs=[pl.BlockSpec((2, N_EDGES), lambda: (0, 0))],
        out_specs=[pl.BlockSpec((N_EDGES,), lambda: (0,)),
                   pl.BlockSpec((N_EDGES,), lambda: (0,))],
        out_shape=[jax.ShapeDtypeStruct((N_EDGES,), jnp.int32),
                   jax.ShapeDtypeStruct((N_EDGES,), jnp.int32)],
    )(e)
DP1 = 40


def _mm_body(a_ref, w_ref, c_ref, o_ref):
    o_ref[...] = jnp.dot(a_ref[...], w_ref[...],
                         preferred_element_type=jnp.float32) + c_ref[...]


def _tc_matmul(a, w, crow):
    grid = N_NODES // _ROWS
    dk, dn = w.shape
    return pl.pallas_call(
        _mm_body,
        grid=(grid,),
        in_specs=[
            pl.BlockSpec((_ROWS, dk), lambda i: (i, 0)),
            pl.BlockSpec((dk, dn), lambda i: (0, 0)),
            pl.BlockSpec((1, dn), lambda i: (0, 0)),
        ],
        out_specs=pl.BlockSpec((_ROWS, dn), lambda i: (i, 0)),
        out_shape=jax.ShapeDtypeStruct((N_NODES, dn), jnp.float32),
    )(a, w, crow)


def _k2_body(agg_ref, xr_ref, b1_ref, h_ref):
    a = agg_ref[0] + agg_ref[1]
    dinv = 1.0 / jnp.clip(a[:, 32:33], 1.0, None)
    h_ref[...] = jax.nn.relu(a[:, :D_HID] * dinv + b1_ref[...] + xr_ref[...])


def _tc_hidden(agg1, xr, b1_row):
    grid = N_NODES // _ROWS
    return pl.pallas_call(
        _k2_body,
        grid=(grid,),
        in_specs=[
            pl.BlockSpec((NC, _ROWS, 128), lambda i: (0, i, 0)),
            pl.BlockSpec((_ROWS, D_HID), lambda i: (i, 0)),
            pl.BlockSpec((1, D_HID), lambda i: (0, 0)),
        ],
        out_specs=pl.BlockSpec((_ROWS, D_HID), lambda i: (i, 0)),
        out_shape=jax.ShapeDtypeStruct((N_NODES, D_HID), jnp.float32),
    )(agg1, xr, b1_row)


def _k3_body(aggh_ref, agg1_ref, hr_ref, w2l_ref, out_ref):
    deg = jnp.clip(agg1_ref[0][:, 32:33] + agg1_ref[1][:, 32:33], 1.0, None)
    m = (aggh_ref[0][:, :D_HID] + aggh_ref[1][:, :D_HID]) / deg
    out_ref[...] = jnp.dot(m, w2l_ref[...],
                           preferred_element_type=jnp.float32) + hr_ref[...]


def _tc_out(aggh, agg1, hr, w2l):
    grid = N_NODES // _ROWS
    return pl.pallas_call(
        _k3_body,
        grid=(grid,),
        in_specs=[
            pl.BlockSpec((NC, _ROWS, 128), lambda i: (0, i, 0)),
            pl.BlockSpec((NC, _ROWS, 128), lambda i: (0, i, 0)),
            pl.BlockSpec((_ROWS, 128), lambda i: (i, 0)),
            pl.BlockSpec((D_HID, 128), lambda i: (0, 0)),
        ],
        out_specs=pl.BlockSpec((_ROWS, 128), lambda i: (i, 0)),
        out_shape=jax.ShapeDtypeStruct((N_NODES, 128), jnp.float32),
    )(aggh, agg1, hr, w2l)


def kernel(x, edge_index, W1_l, b1, W1_r, W2_l, b2, W2_r):
    src, dst = _tc_edge_split(edge_index.astype(jnp.int32))

    wl_pad = jnp.pad(W1_l, ((0, 0), (0, DP1 - D_HID)))
    ones_row = jnp.zeros((1, DP1), jnp.float32).at[0, 32].set(1.0)
    zrow = jnp.zeros((1, D_HID), jnp.float32)
    zeros_p1 = jnp.zeros((N_PAD, DP1), jnp.float32)
    zeros_p2 = jnp.zeros((N_PAD, D_HID), jnp.float32)

    xp_aug = _tc_matmul(x, wl_pad, ones_row)
    agg1 = _sc_segment_sum(xp_aug, src, dst, zeros_p1, DP1)
    xr = _tc_matmul(x, W1_r, zrow)
    h = _tc_hidden(agg1, xr, b1.reshape(1, D_HID))
    aggh = _sc_segment_sum(h, src, dst, zeros_p2, D_HID)
    hr = _tc_matmul(h, W2_r, b2.reshape(1, 128))
    out = _tc_out(aggh, agg1, hr, W2_l)
    return out


# speedup vs baseline: 26.7303x; 1.0011x over previous
import functools

import jax
import jax.numpy as jnp
from jax import lax
from jax.experimental import pallas as pl
from jax.experimental.pallas import tpu as pltpu
from jax.experimental.pallas import tpu_sc as plsc

N_NODES = 10000
N_EDGES = 320000
D_IN = 128
D_HID = 32

NC = 2
NS = 16
NW = NC * NS
CHUNK = 80
NCHUNK = 125
N_PAD = 10240
RPS = N_PAD // NS
NBUF = 5


def _sc_segment_sum(table, src1d, dst1d, zeros, d):
    mesh = plsc.VectorSubcoreMesh(core_axis_name="c", subcore_axis_name="s")

    @functools.partial(
        pl.kernel,
        mesh=mesh,
        compiler_params=pltpu.CompilerParams(use_tc_tiling_on_sc=False),
        out_type=jax.ShapeDtypeStruct((NC, N_PAD, 128), jnp.float32),
        scratch_types=[
            pltpu.VMEM((NCHUNK * CHUNK,), jnp.int32),
            pltpu.VMEM((NCHUNK * CHUNK,), jnp.int32),
            [pltpu.VMEM((CHUNK, d), jnp.float32) for _ in range(NBUF)],
            pltpu.VMEM_SHARED((N_PAD, d), jnp.float32),
            pltpu.SemaphoreType.DMA,
            [pltpu.SemaphoreType.DMA for _ in range(NBUF)],
            [pltpu.SemaphoreType.DMA for _ in range(NBUF)],
        ],
    )
    def k(tab_hbm, src_hbm, dst_hbm, z_hbm, out_hbm,
          srcb, dstb, rows, acc, s_misc, sg, ss):
        cid = lax.axis_index("c")
        sid = lax.axis_index("s")
        wid = sid * NC + cid
        base = wid * NCHUNK * CHUNK

        pltpu.async_copy(src_hbm.at[pl.ds(base, NCHUNK * CHUNK)], srcb, s_misc)
        pltpu.async_copy(dst_hbm.at[pl.ds(base, NCHUNK * CHUNK)], dstb, s_misc)
        pltpu.async_copy(z_hbm.at[pl.ds(sid * RPS, RPS)],
                         acc.at[pl.ds(sid * RPS, RPS)], s_misc)
        pltpu.make_async_copy(src_hbm.at[pl.ds(base, NCHUNK * CHUNK)], srcb,
                              s_misc).wait()
        pltpu.make_async_copy(dst_hbm.at[pl.ds(base, NCHUNK * CHUNK)], dstb,
                              s_misc).wait()

        for kk in range(NBUF - 1):
            pltpu.async_copy(tab_hbm.at[srcb.at[pl.ds(kk * CHUNK, CHUNK)]], rows[kk], sg[kk])
        pltpu.make_async_copy(z_hbm.at[pl.ds(sid * RPS, RPS)],
                              acc.at[pl.ds(sid * RPS, RPS)], s_misc).wait()
        plsc.subcore_barrier()

        def step(s, kk):
            pltpu.make_async_copy(tab_hbm.at[srcb.at[pl.ds(s * CHUNK, CHUNK)]], rows[kk],
                                  sg[kk]).wait()
            pltpu.async_copy(rows[kk], acc.at[dstb.at[pl.ds(s * CHUNK, CHUNK)]], ss[kk], add=True)
            kn = (kk + NBUF - 1) % NBUF
            sn = s + NBUF - 1

            @pl.when(sn < NCHUNK)
            def _():
                @pl.when(s >= 1)
                def _():
                    pltpu.make_async_copy(rows[kn], acc.at[dstb.at[pl.ds((s - 1) * CHUNK, CHUNK)]],
                                          ss[kn]).wait()
                pltpu.async_copy(tab_hbm.at[srcb.at[pl.ds(sn * CHUNK, CHUNK)]], rows[kn], sg[kn])

        @pl.loop(0, NCHUNK // NBUF)
        def _(j):
            s0 = j * NBUF
            for kk in range(NBUF):
                step(s0 + kk, kk)

        for kk in range(NBUF):
            s_last = NCHUNK - NBUF + kk
            pltpu.make_async_copy(rows[kk], acc.at[dstb.at[pl.ds(s_last * CHUNK, CHUNK)]],
                                  ss[kk]).wait()

        plsc.subcore_barrier()
        pltpu.sync_copy(acc.at[pl.ds(sid * RPS, RPS)],
                        out_hbm.at[cid].at[pl.ds(sid * RPS, RPS), pl.ds(0, d)])

    return k(table, src1d, dst1d, zeros)


_ROWS = N_NODES


def _esplit_body(e_ref, s_ref, d_ref):
    s_ref[...] = e_ref[0]
    d_ref[...] = e_ref[1]


def _tc_edge_split(e):
    return pl.pallas_call(
        _esplit_body,
        in_specs=[pl.BlockSpec((2, N_EDGES), lambda: (0, 0))],
        out_specs=[pl.BlockSpec((N_EDGES,), lambda: (0,)),
                   pl.BlockSpec((N_EDGES,), lambda: (0,))],
        out_shape=[jax.ShapeDtypeStruct((N_EDGES,), jnp.int32),
                   jax.ShapeDtypeStruct((N_EDGES,), jnp.int32)],
    )(e)
DP1 = 40


def _mm_body(a_ref, w_ref, c_ref, o_ref):
    o_ref[...] = jnp.dot(a_ref[...], w_ref[...],
                         preferred_element_type=jnp.float32) + c_ref[...]


def _tc_matmul(a, w, crow):
    grid = N_NODES // _ROWS
    dk, dn = w.shape
    return pl.pallas_call(
        _mm_body,
        grid=(grid,),
        in_specs=[
            pl.BlockSpec((_ROWS, dk), lambda i: (i, 0)),
            pl.BlockSpec((dk, dn), lambda i: (0, 0)),
            pl.BlockSpec((1, dn), lambda i: (0, 0)),
        ],
        out_specs=pl.BlockSpec((_ROWS, dn), lambda i: (i, 0)),
        out_shape=jax.ShapeDtypeStruct((N_NODES, dn), jnp.float32),
    )(a, w, crow)


def _k2_body(agg_ref, xr_ref, b1_ref, h_ref):
    a = agg_ref[0] + agg_ref[1]
    dinv = 1.0 / jnp.clip(a[:, 32:33], 1.0, None)
    h_ref[...] = jax.nn.relu(a[:, :D_HID] * dinv + b1_ref[...] + xr_ref[...])


def _tc_hidden(agg1, xr, b1_row):
    grid = N_NODES // _ROWS
    return pl.pallas_call(
        _k2_body,
        grid=(grid,),
        in_specs=[
            pl.BlockSpec((NC, _ROWS, 128), lambda i: (0, i, 0)),
            pl.BlockSpec((_ROWS, D_HID), lambda i: (i, 0)),
            pl.BlockSpec((1, D_HID), lambda i: (0, 0)),
        ],
        out_specs=pl.BlockSpec((_ROWS, D_HID), lambda i: (i, 0)),
        out_shape=jax.ShapeDtypeStruct((N_NODES, D_HID), jnp.float32),
    )(agg1, xr, b1_row)


def _k3_body(aggh_ref, agg1_ref, hr_ref, w2l_ref, out_ref):
    deg = jnp.clip(agg1_ref[0][:, 32:33] + agg1_ref[1][:, 32:33], 1.0, None)
    m = (aggh_ref[0][:, :D_HID] + aggh_ref[1][:, :D_HID]) / deg
    out_ref[...] = jnp.dot(m, w2l_ref[...],
                           preferred_element_type=jnp.float32) + hr_ref[...]


def _tc_out(aggh, agg1, hr, w2l):
    grid = N_NODES // _ROWS
    return pl.pallas_call(
        _k3_body,
        grid=(grid,),
        in_specs=[
            pl.BlockSpec((NC, _ROWS, 128), lambda i: (0, i, 0)),
            pl.BlockSpec((NC, _ROWS, 128), lambda i: (0, i, 0)),
            pl.BlockSpec((_ROWS, 128), lambda i: (i, 0)),
            pl.BlockSpec((D_HID, 128), lambda i: (0, 0)),
        ],
        out_specs=pl.BlockSpec((_ROWS, 128), lambda i: (i, 0)),
        out_shape=jax.ShapeDtypeStruct((N_NODES, 128), jnp.float32),
    )(aggh, agg1, hr, w2l)


def kernel(x, edge_index, W1_l, b1, W1_r, W2_l, b2, W2_r):
    src, dst = _tc_edge_split(edge_index.astype(jnp.int32))

    wl_pad = jnp.pad(W1_l, ((0, 0), (0, DP1 - D_HID)))
    ones_row = jnp.zeros((1, DP1), jnp.float32).at[0, 32].set(1.0)
    zrow = jnp.zeros((1, D_HID), jnp.float32)
    zeros_p1 = jnp.zeros((N_PAD, DP1), jnp.float32)
    zeros_p2 = jnp.zeros((N_PAD, D_HID), jnp.float32)

    xp_aug = _tc_matmul(x, wl_pad, ones_row)
    agg1 = _sc_segment_sum(xp_aug, src, dst, zeros_p1, DP1)
    xr = _tc_matmul(x, W1_r, zrow)
    h = _tc_hidden(agg1, xr, b1.reshape(1, D_HID))
    aggh = _sc_segment_sum(h, src, dst, zeros_p2, D_HID)
    hr = _tc_matmul(h, W2_r, b2.reshape(1, 128))
    out = _tc_out(aggh, agg1, hr, W2_l)
    return out
